# baseline (speedup 1.0000x reference)
import math

import jax
import jax.numpy as jnp
from jax import lax
from jax.experimental import pallas as pl
from jax.experimental.pallas import tpu as pltpu
from jax.experimental.pallas import tpu_sc as plsc

VOCAB = 1000000
DIM = 64
ROWS = 4096
COLS = 200
B = ROWS * COLS
SCALE = math.sqrt(DIM)

NC = 2
NS = 16
NW = NC * NS

RCHUNK = 200
N_RCHUNKS = VOCAB // RCHUNK
RCH_PER_W = 158

B_PER_W = B // NW
CHUNK = COLS
N_CHUNKS = B_PER_W // CHUNK
SEQ_PER_W = ROWS // NW


def _repack_kernel(table_hbm, wide_hbm, bufa0, bufa1, bufw0, bufw1, rsem0,
                   rsem1, wsem0, wsem1):
    wid = lax.axis_index("s") * NC + lax.axis_index("c")
    bufas = (bufa0, bufa1)
    bufws = (bufw0, bufw1)
    rsems = (rsem0, rsem1)
    wsems = (wsem0, wsem1)

    def read(c, b):
        pltpu.make_async_copy(
            table_hbm.at[pl.ds(c * RCHUNK, RCHUNK)], bufas[b], rsems[b]
        ).start()

    def read_wait(c, b):
        pltpu.make_async_copy(
            table_hbm.at[pl.ds(c * RCHUNK, RCHUNK)], bufas[b], rsems[b]
        ).wait()

    def write(c, b):
        pltpu.make_async_copy(
            bufws[b], wide_hbm.at[pl.ds(c * RCHUNK, RCHUNK)], wsems[b]
        ).start()

    def write_wait(c, b):
        pltpu.make_async_copy(
            bufws[b], wide_hbm.at[pl.ds(c * RCHUNK, RCHUNK)], wsems[b]
        ).wait()

    def vcopy(b):
        src = bufas[b]
        dst = bufws[b]

        @plsc.parallel_loop(0, RCHUNK, unroll=4)
        def _(r):
            for u in range(DIM // 16):
                sl = pl.ds(u * 16, 16)
                dst[r, sl] = src[r, sl]

    read(wid, 0)

    def body(g, _):
        for b in range(2):
            j = 2 * g + b
            c = wid + j * NW

            @pl.when(c < N_RCHUNKS)
            def _():
                read_wait(c, b)

                @pl.when(c + NW < N_RCHUNKS)
                def _():
                    read(c + NW, 1 - b)

                @pl.when(j >= 2)
                def _():
                    write_wait(c - 2 * NW, b)

                vcopy(b)
                write(c, b)

        return 0

    lax.fori_loop(0, RCH_PER_W // 2, body, 0)

    def drain(g, _):
        for b in range(2):
            c = wid + (2 * g + b) * NW

            @pl.when((c >= N_RCHUNKS - 2 * NW) & (c < N_RCHUNKS))
            def _():
                write_wait(c, b)

        return 0

    lax.fori_loop(0, RCH_PER_W // 2, drain, 0)


def _gather_kernel(wide_hbm, idx_hbm, out_hbm, idx_v, buf0, buf1, pk0, pk1,
                   gsem0, gsem1, wsem0, wsem1):
    wid = lax.axis_index("s") * NC + lax.axis_index("c")
    base = wid * B_PER_W
    pltpu.sync_copy(idx_hbm.at[pl.ds(base, B_PER_W)], idx_v)

    bufs = (buf0, buf1)
    pks = (pk0, pk1)
    gsems = (gsem0, gsem1)
    wsems = (wsem0, wsem1)

    def gather(k, b):
        pltpu.make_async_copy(
            wide_hbm.at[idx_v.at[pl.ds(k * CHUNK, CHUNK)]], bufs[b],
            gsems[b],
        ).start()

    def gather_wait(k, b):
        pltpu.make_async_copy(
            wide_hbm.at[idx_v.at[pl.ds(k * CHUNK, CHUNK)]], bufs[b],
            gsems[b],
        ).wait()

    def scale_pack(b):
        buf = bufs[b]
        pk = pks[b]

        @plsc.parallel_loop(0, CHUNK, unroll=4)
        def _(r):
            for u in range(DIM // 16):
                sl = pl.ds(u * 16, 16)
                pk[0, r, sl] = buf[r, sl] * SCALE

    seq0 = wid * SEQ_PER_W

    def write(k, b):
        pltpu.make_async_copy(
            pks[b], out_hbm.at[pl.ds(seq0 + k, 1)], wsems[b]
        ).start()

    def write_wait(k, b):
        pltpu.make_async_copy(
            pks[b], out_hbm.at[pl.ds(seq0 + k, 1)], wsems[b]
        ).wait()

    gather(0, 0)

    def body(g, _):
        for b in range(2):
            k = 2 * g + b
            gather_wait(k, b)

            @pl.when(k + 1 < N_CHUNKS)
            def _():
                gather(k + 1, 1 - b)

            @pl.when(k >= 2)
            def _():
                write_wait(k - 2, b)

            scale_pack(b)
            write(k, b)
        return 0

    lax.fori_loop(0, N_CHUNKS // 2, body, 0)
    write_wait(N_CHUNKS - 2, 0 if (N_CHUNKS - 2) % 2 == 0 else 1)
    write_wait(N_CHUNKS - 1, 0 if (N_CHUNKS - 1) % 2 == 0 else 1)


@jax.jit
def _emb_call(idx_flat, table):
    mesh = plsc.VectorSubcoreMesh(core_axis_name="c", subcore_axis_name="s")
    repack = pl.kernel(
        _repack_kernel,
        out_type=jax.ShapeDtypeStruct((VOCAB, 128), jnp.float32),
        mesh=mesh,
        scratch_types=[
            pltpu.VMEM((RCHUNK, DIM), jnp.float32),
            pltpu.VMEM((RCHUNK, DIM), jnp.float32),
            pltpu.VMEM((RCHUNK, 128), jnp.float32),
            pltpu.VMEM((RCHUNK, 128), jnp.float32),
            pltpu.SemaphoreType.DMA,
            pltpu.SemaphoreType.DMA,
            pltpu.SemaphoreType.DMA,
            pltpu.SemaphoreType.DMA,
        ],
    )
    wide = repack(table)
    gather = pl.kernel(
        _gather_kernel,
        out_type=jax.ShapeDtypeStruct((ROWS, COLS, DIM), jnp.float32),
        mesh=mesh,
        scratch_types=[
            pltpu.VMEM((B_PER_W,), jnp.int32),
            pltpu.VMEM((CHUNK, 128), jnp.float32),
            pltpu.VMEM((CHUNK, 128), jnp.float32),
            pltpu.VMEM((1, CHUNK, DIM), jnp.float32),
            pltpu.VMEM((1, CHUNK, DIM), jnp.float32),
            pltpu.SemaphoreType.DMA,
            pltpu.SemaphoreType.DMA,
            pltpu.SemaphoreType.DMA,
            pltpu.SemaphoreType.DMA,
        ],
    )
    return gather(wide, idx_flat)


def kernel(token_ids, table):
    idx_flat = token_ids.reshape(-1).astype(jnp.int32)
    return _emb_call(idx_flat, table)


# speedup vs baseline: 1.0005x; 1.0005x over previous
import math

import jax
import jax.numpy as jnp
from jax import lax
from jax.experimental import pallas as pl
from jax.experimental.pallas import tpu as pltpu
from jax.experimental.pallas import tpu_sc as plsc

VOCAB = 1000000
DIM = 64
ROWS = 4096
COLS = 200
B = ROWS * COLS
SCALE = math.sqrt(DIM)

NC = 2
NS = 16
NW = NC * NS

RCHUNK = 200
N_RCHUNKS = VOCAB // RCHUNK
RCH_PER_W = 158

B_PER_W = B // NW
CHUNK = COLS
N_CHUNKS = B_PER_W // CHUNK
SEQ_PER_W = ROWS // NW


PBLK = 4000


def _pad_kernel(t_ref, w_ref):
    w_ref[:, : DIM] = t_ref[...]


def _gather_kernel(wide_hbm, idx_hbm, out_hbm, idx_v, buf0, buf1, pk0, pk1,
                   gsem0, gsem1, wsem0, wsem1):
    wid = lax.axis_index("s") * NC + lax.axis_index("c")
    base = wid * B_PER_W
    pltpu.sync_copy(idx_hbm.at[pl.ds(base, B_PER_W)], idx_v)

    bufs = (buf0, buf1)
    pks = (pk0, pk1)
    gsems = (gsem0, gsem1)
    wsems = (wsem0, wsem1)

    def gather(k, b):
        pltpu.make_async_copy(
            wide_hbm.at[idx_v.at[pl.ds(k * CHUNK, CHUNK)]], bufs[b],
            gsems[b],
        ).start()

    def gather_wait(k, b):
        pltpu.make_async_copy(
            wide_hbm.at[idx_v.at[pl.ds(k * CHUNK, CHUNK)]], bufs[b],
            gsems[b],
        ).wait()

    def scale_pack(b):
        buf = bufs[b]
        pk = pks[b]

        @plsc.parallel_loop(0, CHUNK, unroll=4)
        def _(r):
            for u in range(DIM // 16):
                sl = pl.ds(u * 16, 16)
                pk[0, r, sl] = buf[r, sl] * SCALE

    seq0 = wid * SEQ_PER_W

    def write(k, b):
        pltpu.make_async_copy(
            pks[b], out_hbm.at[pl.ds(seq0 + k, 1)], wsems[b]
        ).start()

    def write_wait(k, b):
        pltpu.make_async_copy(
            pks[b], out_hbm.at[pl.ds(seq0 + k, 1)], wsems[b]
        ).wait()

    gather(0, 0)

    def body(g, _):
        for b in range(2):
            k = 2 * g + b
            gather_wait(k, b)

            @pl.when(k + 1 < N_CHUNKS)
            def _():
                gather(k + 1, 1 - b)

            @pl.when(k >= 2)
            def _():
                write_wait(k - 2, b)

            scale_pack(b)
            write(k, b)
        return 0

    lax.fori_loop(0, N_CHUNKS // 2, body, 0)
    write_wait(N_CHUNKS - 2, 0 if (N_CHUNKS - 2) % 2 == 0 else 1)
    write_wait(N_CHUNKS - 1, 0 if (N_CHUNKS - 1) % 2 == 0 else 1)


@jax.jit
def _emb_call(idx_flat, table):
    wide = pl.pallas_call(
        _pad_kernel,
        out_shape=jax.ShapeDtypeStruct((VOCAB, 128), jnp.float32),
        grid=(VOCAB // PBLK,),
        in_specs=[pl.BlockSpec((PBLK, DIM), lambda i: (i, 0))],
        out_specs=pl.BlockSpec((PBLK, 128), lambda i: (i, 0)),
    )(table)
    mesh = plsc.VectorSubcoreMesh(core_axis_name="c", subcore_axis_name="s")
    gather = pl.kernel(
        _gather_kernel,
        out_type=jax.ShapeDtypeStruct((ROWS, COLS, DIM), jnp.float32),
        mesh=mesh,
        scratch_types=[
            pltpu.VMEM((B_PER_W,), jnp.int32),
            pltpu.VMEM((CHUNK, 128), jnp.float32),
            pltpu.VMEM((CHUNK, 128), jnp.float32),
            pltpu.VMEM((1, CHUNK, DIM), jnp.float32),
            pltpu.VMEM((1, CHUNK, DIM), jnp.float32),
            pltpu.SemaphoreType.DMA,
            pltpu.SemaphoreType.DMA,
            pltpu.SemaphoreType.DMA,
            pltpu.SemaphoreType.DMA,
        ],
    )
    return gather(wide, idx_flat)


def kernel(token_ids, table):
    idx_flat = token_ids.reshape(-1).astype(jnp.int32)
    return _emb_call(idx_flat, table)


# speedup vs baseline: 1.1004x; 1.0999x over previous
import math

import jax
import jax.numpy as jnp
from jax import lax
from jax.experimental import pallas as pl
from jax.experimental.pallas import tpu as pltpu
from jax.experimental.pallas import tpu_sc as plsc

VOCAB = 1000000
DIM = 64
ROWS = 4096
COLS = 200
B = ROWS * COLS
SCALE = math.sqrt(DIM)

NC = 2
NS = 16
NW = NC * NS

B_PER_W = B // NW
CHUNK = 200
N_CHUNKS = B_PER_W // CHUNK
NBUF = 4

PBLK = 4000


def _pad_kernel(t_ref, w_ref):
    w_ref[:, : DIM] = t_ref[...]


def _gather_kernel(wide_hbm, idx_hbm, out_hbm, idx_v, buf0, buf1, buf2,
                   buf3, sem0, sem1, sem2, sem3, wsem0, wsem1, wsem2,
                   wsem3):
    wid = lax.axis_index("s") * NC + lax.axis_index("c")
    base = wid * B_PER_W
    pltpu.sync_copy(idx_hbm.at[pl.ds(base, B_PER_W)], idx_v)

    bufs = (buf0, buf1, buf2, buf3)
    gsems = (sem0, sem1, sem2, sem3)
    wsems = (wsem0, wsem1, wsem2, wsem3)

    def gather(k, b):
        pltpu.make_async_copy(
            wide_hbm.at[idx_v.at[pl.ds(k * CHUNK, CHUNK)]], bufs[b],
            gsems[b],
        ).start()

    def gather_wait(k, b):
        pltpu.make_async_copy(
            wide_hbm.at[idx_v.at[pl.ds(k * CHUNK, CHUNK)]], bufs[b],
            gsems[b],
        ).wait()

    def scale(b):
        buf = bufs[b]

        @plsc.parallel_loop(0, CHUNK, unroll=4)
        def _(r):
            for u in range(DIM // 16):
                sl = pl.ds(u * 16, 16)
                buf[r, sl] = buf[r, sl] * SCALE

    def write(k, b):
        pltpu.make_async_copy(
            bufs[b], out_hbm.at[pl.ds(base + k * CHUNK, CHUNK)], wsems[b]
        ).start()

    def write_wait(k, b):
        pltpu.make_async_copy(
            bufs[b], out_hbm.at[pl.ds(base + k * CHUNK, CHUNK)], wsems[b]
        ).wait()

    for p in range(NBUF - 1):
        gather(p, p)

    def body(g, _):
        for b in range(NBUF):
            k = NBUF * g + b
            gather_wait(k, b)
            nb = (b + NBUF - 1) % NBUF

            @pl.when(k + NBUF - 1 < N_CHUNKS)
            def _():
                @pl.when(k >= 1)
                def _():
                    write_wait(k - 1, nb)

                gather(k + NBUF - 1, nb)

            scale(b)
            write(k, b)
        return 0

    lax.fori_loop(0, N_CHUNKS // NBUF, body, 0)
    for k in range(N_CHUNKS - NBUF, N_CHUNKS):
        write_wait(k, k % NBUF)


@jax.jit
def _emb_call(idx_flat, table):
    wide = pl.pallas_call(
        _pad_kernel,
        out_shape=jax.ShapeDtypeStruct((VOCAB, 128), jnp.float32),
        grid=(VOCAB // PBLK,),
        in_specs=[pl.BlockSpec((PBLK, DIM), lambda i: (i, 0))],
        out_specs=pl.BlockSpec((PBLK, 128), lambda i: (i, 0)),
    )(table)
    mesh = plsc.VectorSubcoreMesh(core_axis_name="c", subcore_axis_name="s")
    gather = pl.kernel(
        _gather_kernel,
        out_type=jax.ShapeDtypeStruct((B, 128), jnp.float32),
        mesh=mesh,
        scratch_types=[
            pltpu.VMEM((B_PER_W,), jnp.int32),
            pltpu.VMEM((CHUNK, 128), jnp.float32),
            pltpu.VMEM((CHUNK, 128), jnp.float32),
            pltpu.VMEM((CHUNK, 128), jnp.float32),
            pltpu.VMEM((CHUNK, 128), jnp.float32),
            pltpu.SemaphoreType.DMA,
            pltpu.SemaphoreType.DMA,
            pltpu.SemaphoreType.DMA,
            pltpu.SemaphoreType.DMA,
            pltpu.SemaphoreType.DMA,
            pltpu.SemaphoreType.DMA,
            pltpu.SemaphoreType.DMA,
            pltpu.SemaphoreType.DMA,
        ],
    )
    o2 = gather(wide, idx_flat)
    return o2[:, :DIM].reshape(ROWS, COLS, DIM)


def kernel(token_ids, table):
    idx_flat = token_ids.reshape(-1).astype(jnp.int32)
    return _emb_call(idx_flat, table)


# speedup vs baseline: 1.2746x; 1.1583x over previous
import math

import jax
import jax.numpy as jnp
from jax import lax
from jax.experimental import pallas as pl
from jax.experimental.pallas import tpu as pltpu
from jax.experimental.pallas import tpu_sc as plsc

VOCAB = 1000000
DIM = 64
ROWS = 4096
COLS = 200
B = ROWS * COLS
SCALE = math.sqrt(DIM)

NC = 2
NS = 16
NW = NC * NS

B_PER_W = B // NW
CHUNK = 200
N_CHUNKS = B_PER_W // CHUNK
NBUF = 4

def _gather_kernel(wide_hbm, idx_hbm, out_hbm, idx_v, buf0, buf1, buf2,
                   buf3, sem0, sem1, sem2, sem3, wsem0, wsem1, wsem2,
                   wsem3):
    wid = lax.axis_index("s") * NC + lax.axis_index("c")
    base = wid * B_PER_W
    pltpu.sync_copy(idx_hbm.at[pl.ds(base, B_PER_W)], idx_v)

    bufs = (buf0, buf1, buf2, buf3)
    gsems = (sem0, sem1, sem2, sem3)
    wsems = (wsem0, wsem1, wsem2, wsem3)

    def gather(k, b):
        pltpu.make_async_copy(
            wide_hbm.at[idx_v.at[pl.ds(k * CHUNK, CHUNK)]], bufs[b],
            gsems[b],
        ).start()

    def gather_wait(k, b):
        pltpu.make_async_copy(
            wide_hbm.at[idx_v.at[pl.ds(k * CHUNK, CHUNK)]], bufs[b],
            gsems[b],
        ).wait()

    def scale(b):
        buf = bufs[b]

        @plsc.parallel_loop(0, CHUNK, unroll=4)
        def _(r):
            for u in range(DIM // 16):
                sl = pl.ds(u * 16, 16)
                buf[r, sl] = buf[r, sl] * SCALE

    def write(k, b):
        pltpu.make_async_copy(
            bufs[b], out_hbm.at[pl.ds(base + k * CHUNK, CHUNK)], wsems[b]
        ).start()

    def write_wait(k, b):
        pltpu.make_async_copy(
            bufs[b], out_hbm.at[pl.ds(base + k * CHUNK, CHUNK)], wsems[b]
        ).wait()

    for p in range(NBUF - 1):
        gather(p, p)

    def body(g, _):
        for b in range(NBUF):
            k = NBUF * g + b
            gather_wait(k, b)
            nb = (b + NBUF - 1) % NBUF

            @pl.when(k + NBUF - 1 < N_CHUNKS)
            def _():
                @pl.when(k >= 1)
                def _():
                    write_wait(k - 1, nb)

                gather(k + NBUF - 1, nb)

            scale(b)
            write(k, b)
        return 0

    lax.fori_loop(0, N_CHUNKS // NBUF, body, 0)
    for k in range(N_CHUNKS - NBUF, N_CHUNKS):
        write_wait(k, k % NBUF)


@jax.jit
def _emb_call(idx_flat, table):
    wide = jnp.pad(table, ((0, 0), (0, 128 - DIM)))
    mesh = plsc.VectorSubcoreMesh(core_axis_name="c", subcore_axis_name="s")
    gather = pl.kernel(
        _gather_kernel,
        out_type=jax.ShapeDtypeStruct((B, 128), jnp.float32),
        mesh=mesh,
        scratch_types=[
            pltpu.VMEM((B_PER_W,), jnp.int32),
            pltpu.VMEM((CHUNK, 128), jnp.float32),
            pltpu.VMEM((CHUNK, 128), jnp.float32),
            pltpu.VMEM((CHUNK, 128), jnp.float32),
            pltpu.VMEM((CHUNK, 128), jnp.float32),
            pltpu.SemaphoreType.DMA,
            pltpu.SemaphoreType.DMA,
            pltpu.SemaphoreType.DMA,
            pltpu.SemaphoreType.DMA,
            pltpu.SemaphoreType.DMA,
            pltpu.SemaphoreType.DMA,
            pltpu.SemaphoreType.DMA,
            pltpu.SemaphoreType.DMA,
        ],
    )
    o2 = gather(wide, idx_flat)
    return o2[:, :DIM].reshape(ROWS, COLS, DIM)


def kernel(token_ids, table):
    idx_flat = token_ids.reshape(-1).astype(jnp.int32)
    return _emb_call(idx_flat, table)
